# Initial kernel scaffold; baseline (speedup 1.0000x reference)
import jax
import jax.numpy as jnp
from jax.experimental import pallas as pl


def kernel(uids, iids, u_item_pad, u_user_pad, u_user_item_pad, i_user_pad, params):
    raise NotImplementedError("write your pallas kernel here")


# speedup vs baseline: 1.0346x; 1.0346x over previous
import functools

import jax
import jax.numpy as jnp
import numpy as np
from jax import lax
from jax.experimental import pallas as pl
from jax.experimental.pallas import tpu as pltpu
from jax.experimental.pallas import tpu_sc as plsc

D = 64
B = 1024
L = 50
NB = 20
NBI = 20
EPS = 1e-10

BB = 8
NSTEPS = B // BB
NW = 32
CHUNK = 128


def _sc_gather(table, idx_flat):
    n = idx_flat.shape[0]
    n_pad = -(-n // (NW * CHUNK)) * (NW * CHUNK)
    idx_p = jnp.pad(idx_flat.astype(jnp.int32), (0, n_pad - n))
    per_w = n_pad // NW
    n_ch = per_w // CHUNK
    mesh = plsc.VectorSubcoreMesh(core_axis_name="c", subcore_axis_name="s")

    @functools.partial(
        pl.kernel,
        out_type=jax.ShapeDtypeStruct((n_pad, D), jnp.float32),
        mesh=mesh,
        scratch_types=[
            pltpu.VMEM((CHUNK,), jnp.int32),
            pltpu.VMEM((CHUNK, D), jnp.float32),
            pltpu.SemaphoreType.DMA,
        ],
        compiler_params=pltpu.CompilerParams(use_tc_tiling_on_sc=False),
    )
    def gk(table_h, idx_h, out_h, idx_v, rows_v, sem):
        wid = lax.axis_index("s") * 2 + lax.axis_index("c")
        base = wid * per_w

        def step(j, carry):
            off = base + j * CHUNK
            pltpu.sync_copy(idx_h.at[pl.ds(off, CHUNK)], idx_v)
            pltpu.async_copy(table_h.at[idx_v], rows_v, sem).wait()
            pltpu.sync_copy(rows_v, out_h.at[pl.ds(off, CHUNK)])
            return carry

        lax.fori_loop(0, n_ch, step, 0)

    return gk(table, idx_p)[:n]


def _lin_prep(p):
    return {"w": p["W"].T, "b": p["b"][None, :]}


def _mlp_prep(p, out1):
    w1t = p["l1"]["W"].T
    d = {"wa": w1t[:D], "wb": w1t[D:], "b1": p["l1"]["b"][None, :]}
    if out1:
        d["w2"] = p["l2"]["W"]
        d["b2"] = p["l2"]["b"][None, :]
    else:
        d["w2"] = p["l2"]["W"].T
        d["b2"] = p["l2"]["b"][None, :]
    return d


def _comb_prep(p):
    def fold(lp, bp):
        s = bp["gamma"] / jnp.sqrt(bp["var"] + 1e-5)
        return lp["W"].T * s[None, :], ((lp["b"] - bp["mean"]) * s + bp["beta"])[None, :]

    w1, t1 = fold(p["l1"], p["bn1"])
    w2, t2 = fold(p["l2"], p["bn2"])
    return {"w1a": w1[:D], "w1b": w1[D:2 * D], "w1c": w1[2 * D:], "t1": t1,
            "w2": w2, "t2": t2, "w3": p["l3"]["W"].T, "b3": p["l3"]["b"][None, :]}


def _prep_params(params):
    pi, pu, pr = params["item"], params["user"], params["rate"]
    item = {"w1": _lin_prep(pi["w1"]),
            "g_u": _mlp_prep(pi["g_u"], False),
            "g_v": _mlp_prep(pi["g_v"], False),
            "att_i": _mlp_prep(pi["att_i"], True),
            "uia": _mlp_prep(pi["user_items_att"], True),
            "agg_u": _lin_prep(pi["aggre_users_i"]["l"]),
            "agg_i": _lin_prep(pi["aggre_items"]["l"]),
            "comb": _comb_prep(pi["combine"])}
    user = {"w1": _lin_prep(pu["w1"]), "w4": _lin_prep(pu["w4"]),
            "w5": _lin_prep(pu["w5"]), "w6": _lin_prep(pu["w6"]),
            "g_v": _mlp_prep(pu["g_v"], False),
            "uia": _mlp_prep(pu["user_items_att"], True),
            "agg_i": _lin_prep(pu["aggre_items"]["l"]),
            "att_s1": _mlp_prep(pu["user_items_att_s1"], True),
            "agg_s1": _lin_prep(pu["aggre_items_s1"]["l"]),
            "uuu": _mlp_prep(pu["u_user_users_att"], True),
            "agg_n": _lin_prep(pu["u_aggre_neigbors"]["l"]),
            "att_s2": _mlp_prep(pu["user_users_att_s2"], True),
            "agg_n2": _lin_prep(pu["aggre_neigbors_s2"]["l"]),
            "comb": _comb_prep(pu["combine"])}
    w1t = pr["l1"]["W"].T
    rate = {"wa": w1t[:D], "wb": w1t[D:], "b1": pr["l1"]["b"][None, :],
            "w2": pr["l2"]["W"], "b2": pr["l2"]["b"][None, :]}
    return {"item": item, "user": user, "rate": rate}


def _consts():
    eye = np.eye
    ones = np.ones
    bdl = np.kron(eye(BB, dtype=np.float32), ones((1, L), np.float32))
    bdnb = np.kron(eye(BB, dtype=np.float32), ones((1, NB), np.float32))
    bdnbi = np.kron(eye(BB * NB, dtype=np.float32), ones((1, NBI), np.float32))
    return {"BDL": jnp.asarray(bdl), "BDLT": jnp.asarray(bdl.T),
            "BDNB": jnp.asarray(bdnb), "BDNBT": jnp.asarray(bdnb.T),
            "BDNBI": jnp.asarray(bdnbi), "BDNBIT": jnp.asarray(bdnbi.T)}


def _mm(a, b):
    return jnp.dot(a, b, preferred_element_type=jnp.float32,
                   precision=lax.Precision.HIGHEST)


def _body(xr, cr, o_ref):
    g = {k: v[...] for k, v in xr.items()}
    c = jax.tree.map(lambda r: r[...], cr)
    prm = c["prm"]

    relu = lambda x: jnp.maximum(x, 0.0)

    def aff(x, l):
        return _mm(x, l["w"]) + l["b"]

    def mlp_d(a, b, q):
        h = _mm(a, q["wa"]) + _mm(b, q["wb"]) + q["b1"]
        h = jnp.where(h >= 0, h, 0.2 * h)
        return _mm(h, q["w2"]) + q["b2"]

    def mlp_1(a, b, q):
        h = _mm(a, q["wa"]) + _mm(b, q["wb"]) + q["b1"]
        h = jnp.where(h >= 0, h, 0.2 * h)
        return jnp.sum(h * q["w2"], axis=-1, keepdims=True) + q["b2"]

    def attagg(e, x, bd):
        return _mm(bd, e * x) / (_mm(bd, e) + EPS)

    m_iu = (g["m_iu"] > 0).astype(jnp.float32)
    m_ui = (g["m_ui"] > 0).astype(jnp.float32)
    m_s = (g["m_s"] > 0).astype(jnp.float32)
    m_su = (g["m_uu"] > 0).astype(jnp.float32)

    pi = prm["item"]
    w1 = pi["w1"]
    f_jt = mlp_d(g["UL"], g["RL1"], pi["g_u"])
    w1p = aff(g["UL"], w1)
    w1q = m_iu * _mm(c["BDLT"], _mm(g["IB"], w1["w"])) + w1["b"]
    e = jnp.exp(mlp_1(w1p, w1q, pi["att_i"])) * m_iu
    z_j = relu(aff(attagg(e, aff(f_jt, w1), c["BDL"]), pi["agg_u"]))

    x_ia = mlp_d(g["IL"], g["RL2"], pi["g_v"])
    w1x = aff(x_ia, w1)
    w1pi = m_ui * _mm(c["BDLT"], _mm(g["UB"], w1["w"])) + w1["b"]
    e = jnp.exp(mlp_1(w1x, w1pi, pi["uia"])) * m_ui
    h_ii = relu(aff(attagg(e, x_ia, c["BDL"]), pi["agg_i"]))

    def comb(a, b, cc, q):
        y = relu(_mm(a, q["w1a"]) + _mm(b, q["w1b"]) + _mm(cc, q["w1c"]) + q["t1"])
        y = relu(_mm(y, q["w2"]) + q["t2"])
        return _mm(y, q["w3"]) + q["b3"]

    z = comb(z_j * h_ii, z_j, h_ii, pi["comb"])

    pu = prm["user"]
    w1u = pu["w1"]
    x_ia_u = mlp_d(g["IL"], g["RL2"], pu["g_v"])
    w1x = aff(x_ia_u, w1u)
    w1pi = m_ui * _mm(c["BDLT"], _mm(g["UB"], w1u["w"])) + w1u["b"]
    e = jnp.exp(mlp_1(w1x, w1pi, pu["uia"])) * m_ui
    h_ii_u = relu(aff(attagg(e, x_ia_u, c["BDL"]), pu["agg_i"]))

    x_s = mlp_d(g["IS"], g["RS"], pu["g_v"])
    w4 = pu["w4"]
    w4x = aff(x_s, w4)
    w4p = m_s * _mm(c["BDNBIT"], _mm(g["UNB"], w4["w"])) + w4["b"]
    e_s = jnp.exp(mlp_1(w4x, w4p, pu["att_s1"])) * m_s
    h_oi = relu(aff(attagg(e_s, x_s, c["BDNBI"]), pu["agg_s1"]))

    w5 = pu["w5"]
    w5h = aff(h_oi, w5)
    w5p = m_su * _mm(c["BDNBT"], _mm(g["UB"], w5["w"])) + w5["b"]
    e_b = jnp.exp(mlp_1(w5h, w5p, pu["uuu"])) * m_su
    h_is1 = relu(aff(attagg(e_b, h_oi, c["BDNB"]), pu["agg_n"]))

    q_n = m_su * g["UNB"]
    w6 = pu["w6"]
    w6q = aff(q_n, w6)
    w6p = m_su * _mm(c["BDNBT"], _mm(g["UB"], w6["w"])) + w6["b"]
    e_b2 = jnp.exp(mlp_1(w6q, w6p, pu["att_s2"])) * m_su
    h_is2 = relu(aff(attagg(e_b2, q_n, c["BDNB"]), pu["agg_n2"]))

    h = comb(h_ii_u, h_is1, h_is2, pu["comb"])

    pr = prm["rate"]
    r = relu(_mm(h, pr["wa"]) + _mm(z, pr["wb"]) + pr["b1"])
    o_ref[...] = jnp.sum(r * pr["w2"], axis=-1, keepdims=True) + pr["b2"]


def _tc_forward(x_in, c_in):
    def x_spec(v):
        return pl.BlockSpec((v.shape[0] // NSTEPS,) + v.shape[1:],
                            lambda i, nd=v.ndim: (i,) + (0,) * (nd - 1))

    def c_spec(v):
        return pl.BlockSpec(v.shape, lambda i, nd=v.ndim: (0,) * nd)

    x_specs = {k: x_spec(v) for k, v in x_in.items()}
    c_specs = jax.tree.map(c_spec, c_in)
    return pl.pallas_call(
        _body,
        grid=(NSTEPS,),
        in_specs=[x_specs, c_specs],
        out_specs=pl.BlockSpec((BB, 1), lambda i: (i, 0)),
        out_shape=jax.ShapeDtypeStruct((B, 1), jnp.float32),
    )(x_in, c_in)


def kernel(uids, iids, u_item_pad, u_user_pad, u_user_item_pad, i_user_pad, params):
    iu_idx = i_user_pad[:, :, 0].reshape(-1)
    iu_r = i_user_pad[:, :, 1].reshape(-1)
    ui_idx = u_item_pad[:, :, 0].reshape(-1)
    ui_r = u_item_pad[:, :, 1].reshape(-1)
    s_idx = u_user_item_pad[..., 0].reshape(-1)
    s_r = u_user_item_pad[..., 1].reshape(-1)
    uu_idx = u_user_pad.reshape(-1)

    urows = _sc_gather(params["user_emb"], jnp.concatenate([iu_idx, uu_idx, uids]))
    irows = _sc_gather(params["item_emb"], jnp.concatenate([ui_idx, s_idx, iids]))
    rrows = _sc_gather(params["rate_emb"], jnp.concatenate([iu_r, ui_r, s_r]))

    nl, ns, nnb = B * L, B * NB * NBI, B * NB
    x_in = {
        "UL": urows[:nl], "UNB": urows[nl:nl + nnb], "UB": urows[nl + nnb:],
        "IL": irows[:nl], "IS": irows[nl:nl + ns], "IB": irows[nl + ns:],
        "RL1": rrows[:nl], "RL2": rrows[nl:2 * nl], "RS": rrows[2 * nl:],
        "m_iu": iu_idx[:, None], "m_ui": ui_idx[:, None],
        "m_s": s_idx[:, None], "m_uu": uu_idx[:, None],
    }
    c_in = dict(_consts())
    c_in["prm"] = _prep_params(params)
    return _tc_forward(x_in, c_in)


# speedup vs baseline: 2.6590x; 2.5700x over previous
import functools

import jax
import jax.numpy as jnp
import numpy as np
from jax import lax
from jax.experimental import pallas as pl
from jax.experimental.pallas import tpu as pltpu
from jax.experimental.pallas import tpu_sc as plsc

D = 64
B = 1024
L = 50
NB = 20
NBI = 20
EPS = 1e-10

BB = 16
NSTEPS = B // BB
NW = 32
CHUNK = 128


def _sc_gather(table, idx_flat):
    n = idx_flat.shape[0]
    n_pad = -(-n // (NW * CHUNK)) * (NW * CHUNK)
    idx_p = jnp.pad(idx_flat.astype(jnp.int32), (0, n_pad - n))
    per_w = n_pad // NW
    n_ch = per_w // CHUNK
    mesh = plsc.VectorSubcoreMesh(core_axis_name="c", subcore_axis_name="s")

    @functools.partial(
        pl.kernel,
        out_type=jax.ShapeDtypeStruct((n_pad, D), jnp.float32),
        mesh=mesh,
        scratch_types=[
            pltpu.VMEM((CHUNK,), jnp.int32),
            pltpu.VMEM((CHUNK, D), jnp.float32),
            pltpu.SemaphoreType.DMA,
        ],
        compiler_params=pltpu.CompilerParams(use_tc_tiling_on_sc=False),
    )
    def gk(table_h, idx_h, out_h, idx_v, rows_v, sem):
        wid = lax.axis_index("s") * 2 + lax.axis_index("c")
        base = wid * per_w

        def step(j, carry):
            off = base + j * CHUNK
            pltpu.sync_copy(idx_h.at[pl.ds(off, CHUNK)], idx_v)
            pltpu.async_copy(table_h.at[idx_v], rows_v, sem).wait()
            pltpu.sync_copy(rows_v, out_h.at[pl.ds(off, CHUNK)])
            return carry

        lax.fori_loop(0, n_ch, step, 0)

    return gk(table, idx_p)


def _att_prep(att, t):
    w1t = att["l1"]["W"].T
    wa, wb = w1t[:D], w1t[D:]
    tw, tb = t["W"].T, t["b"][None, :]
    return {"wa": tw @ wa, "sb": tw @ wb,
            "c": tb @ wa + tb @ wb + att["l1"]["b"][None, :],
            "w2": att["l2"]["W"], "b2": att["l2"]["b"][None, :]}


def _lin_prep(p):
    return {"w": p["W"].T, "b": p["b"][None, :]}


def _comb_prep(p):
    def fold(lp, bp):
        s = bp["gamma"] / jnp.sqrt(bp["var"] + 1e-5)
        return lp["W"].T * s[None, :], ((lp["b"] - bp["mean"]) * s + bp["beta"])[None, :]

    w1, t1 = fold(p["l1"], p["bn1"])
    w2, t2 = fold(p["l2"], p["bn2"])
    return {"w1a": w1[:D], "w1b": w1[D:2 * D], "w1c": w1[2 * D:], "t1": t1,
            "w2": w2, "t2": t2, "w3": p["l3"]["W"].T, "b3": p["l3"]["b"][None, :]}


def _prep_params(params):
    pi, pu, pr = params["item"], params["user"], params["rate"]
    z64 = jnp.zeros((D, D), jnp.float32)

    W1gu, b1gu = pi["g_u"]["l1"]["W"].T, pi["g_u"]["l1"]["b"][None, :]
    W2gu, b2gu = pi["g_u"]["l2"]["W"].T, pi["g_u"]["l2"]["b"][None, :]
    atti = _att_prep(pi["att_i"], pi["w1"])
    WA = jnp.concatenate(
        [W1gu, jnp.concatenate([atti["wa"], z64], 0)], 1)

    W1vi, b1vi = pi["g_v"]["l1"]["W"].T, pi["g_v"]["l1"]["b"][None, :]
    W2vi, b2vi = pi["g_v"]["l2"]["W"].T, pi["g_v"]["l2"]["b"][None, :]
    W1vu, b1vu = pu["g_v"]["l1"]["W"].T, pu["g_v"]["l1"]["b"][None, :]
    W2vu, b2vu = pu["g_v"]["l2"]["W"].T, pu["g_v"]["l2"]["b"][None, :]
    WB = jnp.concatenate([W1vi, W1vu], 1)
    b1v = jnp.concatenate([b1vi, b1vu], 1)
    uit = _att_prep(pi["user_items_att"], pi["w1"])
    uus = _att_prep(pu["user_items_att"], pu["w1"])
    M_it, M_us = W2vi @ uit["wa"], W2vu @ uus["wa"]
    uit["c"] = uit["c"] + b2vi @ uit["wa"]
    uus["c"] = uus["c"] + b2vu @ uus["wa"]
    WC = jnp.concatenate(
        [jnp.concatenate([M_it, z64], 1),
         jnp.concatenate([z64, M_us], 1)], 0)

    s1 = _att_prep(pu["user_items_att_s1"], pu["w4"])
    M4 = W2vu @ s1["wa"]
    s1["c"] = s1["c"] + b2vu @ s1["wa"]
    uuu = _att_prep(pu["u_user_users_att"], pu["w5"])
    s2 = _att_prep(pu["user_users_att_s2"], pu["w6"])

    return {
        "WA": WA, "b1gu": b1gu, "W2gu": W2gu, "b2gu": b2gu,
        "w1": _lin_prep(pi["w1"]), "atti": atti,
        "aggU": _lin_prep(pi["aggre_users_i"]["l"]),
        "WB": WB, "b1v": b1v, "WC": WC,
        "uit": uit, "W2vi": W2vi, "b2vi": b2vi,
        "aggIit": _lin_prep(pi["aggre_items"]["l"]),
        "combI": _comb_prep(pi["combine"]),
        "uus": uus, "W2vu": W2vu, "b2vu": b2vu,
        "aggIus": _lin_prep(pu["aggre_items"]["l"]),
        "WD": W1vu, "b1vu": b1vu, "M4": M4, "s1": s1,
        "aggS1": _lin_prep(pu["aggre_items_s1"]["l"]),
        "uuu": uuu, "aggN": _lin_prep(pu["u_aggre_neigbors"]["l"]),
        "s2": s2, "aggN2": _lin_prep(pu["aggre_neigbors_s2"]["l"]),
        "combU": _comb_prep(pu["combine"]),
        "rate": {"wt": pr["l1"]["W"].T, "b1": pr["l1"]["b"][None, :],
                 "w2": pr["l2"]["W"], "b2": pr["l2"]["b"][None, :]},
    }


def _mm(a, b):
    return jnp.dot(a, b, preferred_element_type=jnp.float32,
                   precision=lax.Precision.HIGHEST)


def _mm3(a, b):
    d = lambda x, y: jnp.dot(x, y, preferred_element_type=jnp.float32,
                             precision=lax.Precision.DEFAULT)
    a_hi = a.astype(jnp.bfloat16).astype(jnp.float32)
    b_hi = b.astype(jnp.bfloat16).astype(jnp.float32)
    return d(a_hi, b_hi) + d(a - a_hi, b_hi) + d(a_hi, b - b_hi)


def _body(xr, cr, o_ref):
    g = {k: v[...] for k, v in xr.items()}
    p = jax.tree.map(lambda r: r[...], cr)

    relu = lambda x: jnp.maximum(x, 0.0)
    leaky = lambda x: jnp.where(x >= 0, x, 0.2 * x)

    def ssum(x, seg):
        n, w = x.shape
        return jnp.sum(x.reshape(n // seg, seg, w), 1)

    def expand(x, seg):
        gn, w = x.shape
        return jnp.broadcast_to(x[:, None, :], (gn, seg, w)).reshape(gn * seg, w)

    def logits(ha, q):
        return jnp.sum(ha * q["w2"], -1, keepdims=True) + q["b2"]

    def aff(x, l):
        return _mm(x, l["w"]) + l["b"]

    def comb(a, b, cc, q):
        y = relu(_mm(a, q["w1a"]) + _mm(b, q["w1b"]) + _mm(cc, q["w1c"]) + q["t1"])
        y = relu(_mm(y, q["w2"]) + q["t2"])
        return _mm(y, q["w3"]) + q["b3"]

    m_iu = (g["m_iu"] > 0).astype(jnp.float32)
    m_ui = (g["m_ui"] > 0).astype(jnp.float32)
    m_s = (g["m_s"] > 0).astype(jnp.float32)
    m_su = (g["m_uu"] > 0).astype(jnp.float32)

    HA = _mm3(jnp.concatenate([g["UL"], g["RL1"]], 1), p["WA"])
    hg = leaky(HA[:, :D] + p["b1gu"])
    sm = m_iu * expand(_mm(g["IB"], p["atti"]["sb"]), L)
    ha = leaky(HA[:, D:] + sm + p["atti"]["c"])
    e = jnp.exp(logits(ha, p["atti"])) * m_iu
    den = ssum(e, L)
    num = _mm(_mm(ssum(e * hg, L), p["W2gu"]) + den * p["b2gu"], p["w1"]["w"]) \
        + den * p["w1"]["b"]
    z_j = relu(aff(num / (den + EPS), p["aggU"]))

    Hv = leaky(_mm3(jnp.concatenate([g["IL"], g["RL2"]], 1), p["WB"]) + p["b1v"])
    HC = _mm3(Hv, p["WC"])
    h_vi, h_vu = Hv[:, :D], Hv[:, D:]

    ha = leaky(HC[:, :D] + m_ui * expand(_mm(g["UB"], p["uit"]["sb"]), L)
               + p["uit"]["c"])
    e = jnp.exp(logits(ha, p["uit"])) * m_ui
    den = ssum(e, L)
    num = _mm(ssum(e * h_vi, L), p["W2vi"]) + den * p["b2vi"]
    h_ii_it = relu(aff(num / (den + EPS), p["aggIit"]))
    z = comb(z_j * h_ii_it, z_j, h_ii_it, p["combI"])

    ha = leaky(HC[:, D:] + m_ui * expand(_mm(g["UB"], p["uus"]["sb"]), L)
               + p["uus"]["c"])
    e = jnp.exp(logits(ha, p["uus"])) * m_ui
    den = ssum(e, L)
    num = _mm(ssum(e * h_vu, L), p["W2vu"]) + den * p["b2vu"]
    h_ii_us = relu(aff(num / (den + EPS), p["aggIus"]))

    h_su = leaky(_mm3(jnp.concatenate([g["IS"], g["RS"]], 1), p["WD"]) + p["b1vu"])
    sm = m_s * expand(_mm(g["UNB"], p["s1"]["sb"]), NBI)
    ha = leaky(_mm3(h_su, p["M4"]) + sm + p["s1"]["c"])
    e_s = jnp.exp(logits(ha, p["s1"])) * m_s
    den_s = ssum(e_s, NBI)
    temp = (_mm(ssum(e_s * h_su, NBI), p["W2vu"]) + den_s * p["b2vu"]) \
        / (den_s + EPS)
    h_oi = relu(aff(temp, p["aggS1"]))

    sm = m_su * expand(_mm(g["UB"], p["uuu"]["sb"]), NB)
    ha = leaky(_mm(h_oi, p["uuu"]["wa"]) + sm + p["uuu"]["c"])
    e_b = jnp.exp(logits(ha, p["uuu"])) * m_su
    den_b = ssum(e_b, NB)
    h_is1 = relu(aff(ssum(e_b * h_oi, NB) / (den_b + EPS), p["aggN"]))

    q_n = m_su * g["UNB"]
    sm = m_su * expand(_mm(g["UB"], p["s2"]["sb"]), NB)
    ha = leaky(_mm(q_n, p["s2"]["wa"]) + sm + p["s2"]["c"])
    e_b2 = jnp.exp(logits(ha, p["s2"])) * m_su
    den_b2 = ssum(e_b2, NB)
    h_is2 = relu(aff(ssum(e_b2 * q_n, NB) / (den_b2 + EPS), p["aggN2"]))

    h = comb(h_ii_us, h_is1, h_is2, p["combU"])

    pr = p["rate"]
    r = relu(_mm(jnp.concatenate([h, z], 1), pr["wt"]) + pr["b1"])
    o_ref[...] = jnp.sum(r * pr["w2"], axis=-1, keepdims=True) + pr["b2"]


def _tc_forward(x_in, c_in):
    arrays = {}
    x_specs = {}
    for k, (arr, rows, start) in x_in.items():
        arrays[k] = arr
        sb = start // rows
        x_specs[k] = pl.BlockSpec(
            (rows,) + arr.shape[1:],
            lambda i, sb=sb, nd=arr.ndim: (sb + i,) + (0,) * (nd - 1))

    def c_spec(v):
        return pl.BlockSpec(v.shape, lambda i, nd=v.ndim: (0,) * nd)

    c_specs = jax.tree.map(c_spec, c_in)
    return pl.pallas_call(
        _body,
        grid=(NSTEPS,),
        in_specs=[x_specs, c_specs],
        out_specs=pl.BlockSpec((BB, 1), lambda i: (i, 0)),
        out_shape=jax.ShapeDtypeStruct((B, 1), jnp.float32),
    )(arrays, c_in)


def kernel(uids, iids, u_item_pad, u_user_pad, u_user_item_pad, i_user_pad, params):
    iu_idx = i_user_pad[:, :, 0].reshape(-1)
    iu_r = i_user_pad[:, :, 1].reshape(-1)
    ui_idx = u_item_pad[:, :, 0].reshape(-1)
    ui_r = u_item_pad[:, :, 1].reshape(-1)
    s_idx = u_user_item_pad[..., 0].reshape(-1)
    s_r = u_user_item_pad[..., 1].reshape(-1)
    uu_idx = u_user_pad.reshape(-1)

    urows = _sc_gather(params["user_emb"], jnp.concatenate([iu_idx, uu_idx, uids]))
    irows = _sc_gather(params["item_emb"], jnp.concatenate([ui_idx, s_idx, iids]))
    rrows = _sc_gather(params["rate_emb"], jnp.concatenate([iu_r, ui_r, s_r]))

    nl, ns, nnb = B * L, B * NB * NBI, B * NB
    nlb, nsb, nnbb = BB * L, BB * NB * NBI, BB * NB
    x_in = {
        "UL": (urows, nlb, 0), "UNB": (urows, nnbb, nl), "UB": (urows, BB, nl + nnb),
        "IL": (irows, nlb, 0), "IS": (irows, nsb, nl), "IB": (irows, BB, nl + ns),
        "RL1": (rrows, nlb, 0), "RL2": (rrows, nlb, nl), "RS": (rrows, nsb, 2 * nl),
        "m_iu": (iu_idx[:, None], nlb, 0), "m_ui": (ui_idx[:, None], nlb, 0),
        "m_s": (s_idx[:, None], nsb, 0), "m_uu": (uu_idx[:, None], nnbb, 0),
    }
    return _tc_forward(x_in, _prep_params(params))
